# Initial kernel scaffold; baseline (speedup 1.0000x reference)
import jax
import jax.numpy as jnp
from jax.experimental import pallas as pl


def kernel(node_features, edge_index, block_types, W_proj, b_proj, block_emb_table, Wl, bl, Wr, W_ih_f, W_hh_f, b_ih_f, b_hh_f, W_ih_b, W_hh_b, b_ih_b, b_hh_b, W_branch, b_branch, W_loop, b_loop):
    raise NotImplementedError("write your pallas kernel here")


# speedup vs baseline: 7.7269x; 7.7269x over previous
import jax
import jax.numpy as jnp
from jax import lax
from jax.experimental import pallas as pl
from jax.experimental.pallas import tpu as pltpu
from jax.experimental.pallas import tpu_sc as plsc

N = 10000
E = 320000
D = 128
H = 128
LSTM_H = 64

NC = 2
NS = 16
HN = 5120
NP = 5248
RPT = NP // NS
EPT = E // NS
ECH = 80
NCHUNK = EPT // ECH

CHUNK = 1000
GRID = N // CHUNK


def _make_agg(with_deg):
    out_type = [jax.ShapeDtypeStruct((NC, NS, RPT, D), jnp.float32)]
    scratch = [
        pltpu.VMEM((ECH,), jnp.int32),
        pltpu.VMEM((ECH,), jnp.int32),
        pltpu.VMEM((ECH,), jnp.int32),
        pltpu.VMEM((ECH, D), jnp.float32),
        pltpu.VMEM((RPT, D), jnp.float32),
        pltpu.VMEM_SHARED((NP, D), jnp.float32),
        pltpu.SemaphoreType.DMA,
    ]
    if with_deg:
        out_type.append(jax.ShapeDtypeStruct((NC, NS, RPT, D), jnp.float32))
        scratch.append(pltpu.VMEM((ECH, D), jnp.float32))

    mesh = plsc.VectorSubcoreMesh(
        core_axis_name="c", subcore_axis_name="s", num_cores=NC,
        num_subcores=NS)

    def body(*refs):
        if with_deg:
            (x_hbm, src_hbm, dst_hbm, zero_hbm, ones_hbm,
             agg_out, deg_out, srcv, dstv, dstw, rows, sbuf, acc, sem,
             onesv) = refs
        else:
            (x_hbm, src_hbm, dst_hbm, zero_hbm,
             agg_out, srcv, dstv, dstw, rows, sbuf, acc, sem) = refs
        c = lax.axis_index("c")
        s = lax.axis_index("s")
        base_node = c * HN
        rbase = s * RPT
        ebase = s * EPT

        pltpu.sync_copy(zero_hbm, sbuf)
        pltpu.sync_copy(sbuf, acc.at[pl.ds(rbase, RPT)])
        if with_deg:
            pltpu.sync_copy(ones_hbm, onesv)
        plsc.subcore_barrier()

        def remap(j):
            off = ebase + j * ECH
            pltpu.sync_copy(dst_hbm.at[pl.ds(off, ECH)], dstv)
            for k in range(ECH // 16):
                v = dstv[pl.ds(k * 16, 16)]
                local = v - base_node
                valid = (local >= 0) & (local < HN)
                dstw[pl.ds(k * 16, 16)] = jnp.where(valid, local, HN)

        def chunk(j, carry):
            off = ebase + j * ECH
            remap(j)
            pltpu.sync_copy(src_hbm.at[pl.ds(off, ECH)], srcv)
            pltpu.async_copy(x_hbm.at[srcv], rows, sem).wait()
            pltpu.sync_copy(rows, acc.at[dstw], add=True)
            return carry

        lax.fori_loop(0, NCHUNK, chunk, 0)
        plsc.subcore_barrier()
        pltpu.sync_copy(acc.at[pl.ds(rbase, RPT)], sbuf)
        pltpu.sync_copy(sbuf, agg_out.at[c, s])

        if with_deg:
            plsc.subcore_barrier()
            pltpu.sync_copy(zero_hbm, sbuf)
            pltpu.sync_copy(sbuf, acc.at[pl.ds(rbase, RPT)])
            plsc.subcore_barrier()

            def dchunk(j, carry):
                remap(j)
                pltpu.sync_copy(onesv, acc.at[dstw], add=True)
                return carry

            lax.fori_loop(0, NCHUNK, dchunk, 0)
            plsc.subcore_barrier()
            pltpu.sync_copy(acc.at[pl.ds(rbase, RPT)], sbuf)
            pltpu.sync_copy(sbuf, deg_out.at[c, s])

    return pl.kernel(body, out_type=out_type, mesh=mesh,
                     scratch_types=scratch)


def _sc_combine(p):
    q = p.reshape(NC, NP, D)
    return jnp.concatenate([q[0, :HN], q[1, : N - HN]])


def _agg_deg(x, src, dst, zeros_rp, ones_rp):
    agg_p, deg_p = _make_agg(True)(x, src, dst, zeros_rp, ones_rp)
    return _sc_combine(agg_p), _sc_combine(deg_p)[:, 0:16]


def _agg(x, src, dst, zeros_rp):
    return _sc_combine(_make_agg(False)(x, src, dst, zeros_rp)[0])


def _full(shape):
    return pl.BlockSpec(shape, lambda i: tuple(0 for _ in shape))


def _proj(nf, bt3, WpT, b2, table_pad):
    def body(nf_ref, bt_ref, wp_ref, b_ref, tp_ref, o_ref):
        x = jnp.dot(nf_ref[...], wp_ref[...],
                    preferred_element_type=jnp.float32) + b_ref[...]
        ids = bt_ref[0]
        oh = (ids == lax.broadcasted_iota(jnp.int32, (CHUNK, 16), 1)
              ).astype(jnp.float32)
        o_ref[...] = x + jnp.dot(oh, tp_ref[...],
                                 preferred_element_type=jnp.float32)

    return pl.pallas_call(
        body,
        grid=(GRID,),
        in_specs=[
            pl.BlockSpec((CHUNK, D), lambda i: (i, 0)),
            pl.BlockSpec((1, CHUNK, 1), lambda i: (i, 0, 0)),
            _full((D, H)),
            _full((1, H)),
            _full((16, H)),
        ],
        out_specs=pl.BlockSpec((CHUNK, H), lambda i: (i, 0)),
        out_shape=jax.ShapeDtypeStruct((N, H), jnp.float32),
    )(nf, bt3, WpT, b2, table_pad)


_SQRT_HALF = 0.7071067811865476


def _gelu_exact(z):
    return 0.5 * z * (1.0 + lax.erf(z * _SQRT_HALF))


def _layer(agg, deg16, x, WlT, WrT, b2):
    def body(a_ref, d_ref, x_ref, wl_ref, wr_ref, b_ref, o_ref):
        degc = d_ref[:, 0:1]
        inv = 1.0 / jnp.maximum(degc, 1.0)
        z = (jnp.dot(a_ref[...] * inv, wl_ref[...],
                     preferred_element_type=jnp.float32)
             + jnp.dot(x_ref[...], wr_ref[...],
                       preferred_element_type=jnp.float32)
             + b_ref[...])
        o_ref[...] = _gelu_exact(z)

    return pl.pallas_call(
        body,
        grid=(GRID,),
        in_specs=[
            pl.BlockSpec((CHUNK, D), lambda i: (i, 0)),
            pl.BlockSpec((CHUNK, 16), lambda i: (i, 0)),
            pl.BlockSpec((CHUNK, H), lambda i: (i, 0)),
            _full((H, H)),
            _full((H, H)),
            _full((1, H)),
        ],
        out_specs=pl.BlockSpec((CHUNK, H), lambda i: (i, 0)),
        out_shape=jax.ShapeDtypeStruct((N, H), jnp.float32),
    )(agg, deg16, x, WlT, WrT, b2)


def _gates(x, WihC, bC):
    def body(x_ref, w_ref, b_ref, o_ref):
        o_ref[...] = jnp.dot(x_ref[...], w_ref[...],
                             preferred_element_type=jnp.float32) + b_ref[...]

    return pl.pallas_call(
        body,
        grid=(GRID,),
        in_specs=[
            pl.BlockSpec((CHUNK, H), lambda i: (i, 0)),
            _full((H, 4 * H)),
            _full((1, 4 * H)),
        ],
        out_specs=pl.BlockSpec((CHUNK, 4 * H), lambda i: (i, 0)),
        out_shape=jax.ShapeDtypeStruct((N, 4 * H), jnp.float32),
    )(x, WihC, bC)


def _lstm(gx, WhhC):
    T = GRID

    def body(gf_ref, gb_ref, whh_ref, hf_ref, hb_ref, st_ref):
        i = pl.program_id(0)

        @pl.when(i == 0)
        def _():
            st_ref[...] = jnp.zeros_like(st_ref)

        W = whh_ref[...]

        def step(t, carry):
            h, cc = carry
            gxf = gf_ref[pl.ds(t, 1), :]
            gxb = gb_ref[pl.ds(CHUNK - 1 - t, 1), :]
            g_all = (jnp.dot(h, W, preferred_element_type=jnp.float32)
                     + jnp.concatenate([gxf, gxb], axis=1))
            sg = 1.0 / (1.0 + jnp.exp(-g_all))
            th = jnp.tanh(g_all)
            i_cat = jnp.concatenate([sg[:, 0:64], sg[:, 256:320]], axis=1)
            f_cat = jnp.concatenate([sg[:, 64:128], sg[:, 320:384]], axis=1)
            g_cat = jnp.concatenate([th[:, 128:192], th[:, 384:448]], axis=1)
            o_cat = jnp.concatenate([sg[:, 192:256], sg[:, 448:512]], axis=1)
            cc = f_cat * cc + i_cat * g_cat
            h = o_cat * jnp.tanh(cc)
            hf_ref[pl.ds(t, 1), :] = h[:, 0:64]
            hb_ref[pl.ds(CHUNK - 1 - t, 1), :] = h[:, 64:128]
            return (h, cc)

        h0 = st_ref[0:1, :]
        c0 = st_ref[1:2, :]
        h, cc = lax.fori_loop(0, CHUNK, step, (h0, c0))
        st_ref[0:1, :] = h
        st_ref[1:2, :] = cc

    return pl.pallas_call(
        body,
        grid=(T,),
        in_specs=[
            pl.BlockSpec((CHUNK, 256), lambda i: (i, 0)),
            pl.BlockSpec((CHUNK, 256), lambda i: (T - 1 - i, 1)),
            _full((H, 4 * H)),
        ],
        out_specs=[
            pl.BlockSpec((CHUNK, LSTM_H), lambda i: (i, 0)),
            pl.BlockSpec((CHUNK, LSTM_H), lambda i: (T - 1 - i, 0)),
        ],
        out_shape=[
            jax.ShapeDtypeStruct((N, LSTM_H), jnp.float32),
            jax.ShapeDtypeStruct((N, LSTM_H), jnp.float32),
        ],
        scratch_shapes=[pltpu.VMEM((2, H), jnp.float32)],
        compiler_params=pltpu.CompilerParams(
            dimension_semantics=("arbitrary",)),
    )(gx, gx, WhhC)


def _heads(x3, hf, hb, WbP, bbP, WlP, blP):
    def body(x_ref, hf_ref, hb_ref, wb_ref, bb_ref, wl_ref, bl_ref,
             xc_ref, xl_ref, g_ref, b_ref, l_ref):
        i = pl.program_id(0)
        xl = jnp.concatenate([hf_ref[...], hb_ref[...]], axis=1)
        xc = x_ref[...] + xl
        xc_ref[...] = xc
        xl_ref[...] = xl

        @pl.when(i == 0)
        def _():
            g_ref[...] = jnp.zeros_like(g_ref)
            b_ref[...] = jnp.zeros_like(b_ref)
            l_ref[...] = jnp.zeros_like(l_ref)

        g_ref[...] += jnp.sum(xc, axis=0, keepdims=True)
        lane = lax.broadcasted_iota(jnp.int32, (CHUNK, H), 1)

        def probs(w, b, k):
            logits = jnp.dot(xc, w, preferred_element_type=jnp.float32) + b
            logits = jnp.where(lane < k, logits, -1e30)
            m = jnp.max(logits, axis=1, keepdims=True)
            e = jnp.where(lane < k, jnp.exp(logits - m), 0.0)
            return e / jnp.sum(e, axis=1, keepdims=True)

        b_ref[...] += jnp.sum(probs(wb_ref[...], bb_ref[...], 3),
                              axis=0, keepdims=True)
        l_ref[...] += jnp.sum(probs(wl_ref[...], bl_ref[...], 4),
                              axis=0, keepdims=True)

        @pl.when(i == GRID - 1)
        def _():
            g_ref[...] *= (1.0 / N)
            b_ref[...] *= (1.0 / N)
            l_ref[...] *= (1.0 / N)

    return pl.pallas_call(
        body,
        grid=(GRID,),
        in_specs=[
            pl.BlockSpec((CHUNK, H), lambda i: (i, 0)),
            pl.BlockSpec((CHUNK, LSTM_H), lambda i: (i, 0)),
            pl.BlockSpec((CHUNK, LSTM_H), lambda i: (i, 0)),
            _full((H, H)),
            _full((1, H)),
            _full((H, H)),
            _full((1, H)),
        ],
        out_specs=[
            pl.BlockSpec((CHUNK, H), lambda i: (i, 0)),
            pl.BlockSpec((CHUNK, H), lambda i: (i, 0)),
            pl.BlockSpec((1, H), lambda i: (0, 0)),
            pl.BlockSpec((1, H), lambda i: (0, 0)),
            pl.BlockSpec((1, H), lambda i: (0, 0)),
        ],
        out_shape=[
            jax.ShapeDtypeStruct((N, H), jnp.float32),
            jax.ShapeDtypeStruct((N, H), jnp.float32),
            jax.ShapeDtypeStruct((1, H), jnp.float32),
            jax.ShapeDtypeStruct((1, H), jnp.float32),
            jax.ShapeDtypeStruct((1, H), jnp.float32),
        ],
        compiler_params=pltpu.CompilerParams(
            dimension_semantics=("arbitrary",)),
    )(x3, hf, hb, WbP, bbP, WlP, blP)


def kernel(node_features, edge_index, block_types, W_proj, b_proj,
           block_emb_table, Wl, bl, Wr, W_ih_f, W_hh_f, b_ih_f, b_hh_f,
           W_ih_b, W_hh_b, b_ih_b, b_hh_b, W_branch, b_branch, W_loop,
           b_loop):
    f32 = jnp.float32
    src = edge_index[0]
    dst = edge_index[1]

    emb_dim = block_emb_table.shape[1]
    table_pad = jnp.pad(block_emb_table,
                        ((0, 16 - block_emb_table.shape[0]),
                         (0, H - emb_dim)))
    bt3 = block_types.reshape(GRID, CHUNK, 1)
    zeros_rp = jnp.zeros((RPT, D), f32)
    ones_rp = jnp.ones((ECH, D), f32)

    x = _proj(node_features, bt3, W_proj.T, b_proj[None], table_pad)

    agg, deg16 = _agg_deg(x, src, dst, zeros_rp, ones_rp)
    x = _layer(agg, deg16, x, Wl[0].T, Wr[0].T, bl[0][None])
    for l in range(1, Wl.shape[0]):
        agg = _agg(x, src, dst, zeros_rp)
        x = _layer(agg, deg16, x, Wl[l].T, Wr[l].T, bl[l][None])

    WihC = jnp.concatenate([W_ih_f.T, W_ih_b.T], axis=1)
    bC = jnp.concatenate([b_ih_f + b_hh_f, b_ih_b + b_hh_b])[None]
    gx = _gates(x, WihC, bC)

    WhhC = jnp.zeros((H, 4 * H), f32)
    WhhC = WhhC.at[0:LSTM_H, 0:256].set(W_hh_f.T)
    WhhC = WhhC.at[LSTM_H:H, 256:512].set(W_hh_b.T)
    hf, hb = _lstm(gx, WhhC)

    WbP = jnp.zeros((H, H), f32).at[:, 0:3].set(W_branch.T)
    bbP = jnp.zeros((1, H), f32).at[0, 0:3].set(b_branch)
    WlP = jnp.zeros((H, H), f32).at[:, 0:4].set(W_loop.T)
    blP = jnp.zeros((1, H), f32).at[0, 0:4].set(b_loop)

    xc, xl, gsum, bsum, lsum = _heads(x, hf, hb, WbP, bbP, WlP, blP)

    return (xc, gsum, bsum[:, 0:3], lsum[:, 0:4], xl)


# speedup vs baseline: 10.9273x; 1.4142x over previous
import jax
import jax.numpy as jnp
from jax import lax
from jax.experimental import pallas as pl
from jax.experimental.pallas import tpu as pltpu
from jax.experimental.pallas import tpu_sc as plsc

N = 10000
E = 320000
D = 128
H = 128
LSTM_H = 64

NC = 2
NS = 16
HN = 5120
NP = 5248
RPT = NP // NS
EPT = E // NS
ECH = 80
NCHUNK = EPT // ECH

CHUNK = 1000
GRID = N // CHUNK


NBUF = 4
NQUAD = NCHUNK // NBUF
NTAIL = NCHUNK - NQUAD * NBUF


def _make_agg(with_deg):
    out_type = [jax.ShapeDtypeStruct((NC, NS, RPT, D), jnp.float32)]
    scratch = (
        [pltpu.VMEM((ECH,), jnp.int32) for _ in range(NBUF)]
        + [pltpu.VMEM((ECH,), jnp.int32) for _ in range(NBUF)]
        + [pltpu.VMEM((ECH,), jnp.int32) for _ in range(NBUF)]
        + [pltpu.VMEM((ECH, D), jnp.float32) for _ in range(NBUF)]
        + [pltpu.VMEM((RPT, D), jnp.float32)]
        + [pltpu.VMEM_SHARED((NP, D), jnp.float32)]
        + [pltpu.SemaphoreType.DMA for _ in range(NBUF)]
    )
    if with_deg:
        out_type.append(jax.ShapeDtypeStruct((NC, NS, RPT, D), jnp.float32))

    mesh = plsc.VectorSubcoreMesh(
        core_axis_name="c", subcore_axis_name="s", num_cores=NC,
        num_subcores=NS)

    def body(*refs):
        if with_deg:
            x_hbm, src_hbm, dst_hbm, zero_hbm, ones_hbm = refs[:5]
            agg_out, deg_out = refs[5:7]
            rest = refs[7:]
        else:
            x_hbm, src_hbm, dst_hbm, zero_hbm = refs[:4]
            agg_out = refs[4]
            rest = refs[5:]
        srcv = rest[0:NBUF]
        dstv = rest[NBUF:2 * NBUF]
        dstw = rest[2 * NBUF:3 * NBUF]
        rows = rest[3 * NBUF:4 * NBUF]
        sbuf = rest[4 * NBUF]
        acc = rest[4 * NBUF + 1]
        sems = rest[4 * NBUF + 2:4 * NBUF + 2 + NBUF]

        c = lax.axis_index("c")
        s = lax.axis_index("s")
        base_node = c * HN
        rbase = s * RPT
        ebase = s * EPT

        pltpu.sync_copy(zero_hbm, sbuf)
        pltpu.sync_copy(sbuf, acc.at[pl.ds(rbase, RPT)])
        plsc.subcore_barrier()

        def remap(j, b):
            off = ebase + j * ECH
            pltpu.sync_copy(dst_hbm.at[pl.ds(off, ECH)], dstv[b])
            for k in range(ECH // 16):
                v = dstv[b][pl.ds(k * 16, 16)]
                local = v - base_node
                valid = (local >= 0) & (local < HN)
                dstw[b][pl.ds(k * 16, 16)] = jnp.where(valid, local, HN)

        def fetch(j, b):
            off = ebase + j * ECH
            remap(j, b)
            pltpu.sync_copy(src_hbm.at[pl.ds(off, ECH)], srcv[b])
            return pltpu.async_copy(x_hbm.at[srcv[b]], rows[b], sems[b])

        def drain(desc, b):
            desc.wait()
            pltpu.sync_copy(rows[b], acc.at[dstw[b]], add=True)

        def quad(i, carry):
            base = i * NBUF
            descs = [fetch(base + b, b) for b in range(NBUF)]
            for b in range(NBUF):
                drain(descs[b], b)
            return carry

        lax.fori_loop(0, NQUAD, quad, 0)
        descs = [fetch(NQUAD * NBUF + b, b) for b in range(NTAIL)]
        for b in range(NTAIL):
            drain(descs[b], b)

        plsc.subcore_barrier()
        pltpu.sync_copy(acc.at[pl.ds(rbase, RPT)], sbuf)
        pltpu.sync_copy(sbuf, agg_out.at[c, s])

        if with_deg:
            plsc.subcore_barrier()
            pltpu.sync_copy(zero_hbm, sbuf)
            pltpu.sync_copy(sbuf, acc.at[pl.ds(rbase, RPT)])
            pltpu.sync_copy(ones_hbm, rows[0])
            plsc.subcore_barrier()

            def dchunk(j, carry):
                remap(j, 1)
                pltpu.sync_copy(rows[0], acc.at[dstw[1]], add=True)
                return carry

            lax.fori_loop(0, NCHUNK, dchunk, 0)
            plsc.subcore_barrier()
            pltpu.sync_copy(acc.at[pl.ds(rbase, RPT)], sbuf)
            pltpu.sync_copy(sbuf, deg_out.at[c, s])

    return pl.kernel(body, out_type=out_type, mesh=mesh,
                     scratch_types=scratch,
                     compiler_params=pltpu.CompilerParams(
                         use_tc_tiling_on_sc=False))


def _sc_combine(p):
    q = p.reshape(NC, NP, D)
    return jnp.concatenate([q[0, :HN], q[1, : N - HN]])


def _agg_deg(x, src, dst, zeros_rp, ones_rp):
    agg_p, deg_p = _make_agg(True)(x, src, dst, zeros_rp, ones_rp)
    return _sc_combine(agg_p), _sc_combine(deg_p)[:, 0:16]


def _agg(x, src, dst, zeros_rp):
    return _sc_combine(_make_agg(False)(x, src, dst, zeros_rp)[0])


def _full(shape):
    return pl.BlockSpec(shape, lambda i: tuple(0 for _ in shape))


def _proj(nf, bt3, WpT, b2, table_pad):
    def body(nf_ref, bt_ref, wp_ref, b_ref, tp_ref, o_ref):
        x = jnp.dot(nf_ref[...], wp_ref[...],
                    preferred_element_type=jnp.float32) + b_ref[...]
        ids = bt_ref[0]
        oh = (ids == lax.broadcasted_iota(jnp.int32, (CHUNK, 16), 1)
              ).astype(jnp.float32)
        o_ref[...] = x + jnp.dot(oh, tp_ref[...],
                                 preferred_element_type=jnp.float32)

    return pl.pallas_call(
        body,
        grid=(GRID,),
        in_specs=[
            pl.BlockSpec((CHUNK, D), lambda i: (i, 0)),
            pl.BlockSpec((1, CHUNK, 1), lambda i: (i, 0, 0)),
            _full((D, H)),
            _full((1, H)),
            _full((16, H)),
        ],
        out_specs=pl.BlockSpec((CHUNK, H), lambda i: (i, 0)),
        out_shape=jax.ShapeDtypeStruct((N, H), jnp.float32),
    )(nf, bt3, WpT, b2, table_pad)


_SQRT_HALF = 0.7071067811865476


def _gelu_exact(z):
    return 0.5 * z * (1.0 + lax.erf(z * _SQRT_HALF))


def _layer(agg, deg16, x, WlT, WrT, b2):
    def body(a_ref, d_ref, x_ref, wl_ref, wr_ref, b_ref, o_ref):
        degc = d_ref[:, 0:1]
        inv = 1.0 / jnp.maximum(degc, 1.0)
        z = (jnp.dot(a_ref[...] * inv, wl_ref[...],
                     preferred_element_type=jnp.float32)
             + jnp.dot(x_ref[...], wr_ref[...],
                       preferred_element_type=jnp.float32)
             + b_ref[...])
        o_ref[...] = _gelu_exact(z)

    return pl.pallas_call(
        body,
        grid=(GRID,),
        in_specs=[
            pl.BlockSpec((CHUNK, D), lambda i: (i, 0)),
            pl.BlockSpec((CHUNK, 16), lambda i: (i, 0)),
            pl.BlockSpec((CHUNK, H), lambda i: (i, 0)),
            _full((H, H)),
            _full((H, H)),
            _full((1, H)),
        ],
        out_specs=pl.BlockSpec((CHUNK, H), lambda i: (i, 0)),
        out_shape=jax.ShapeDtypeStruct((N, H), jnp.float32),
    )(agg, deg16, x, WlT, WrT, b2)


def _gates(x, WihC, bC):
    def body(x_ref, w_ref, b_ref, o_ref):
        o_ref[...] = jnp.dot(x_ref[...], w_ref[...],
                             preferred_element_type=jnp.float32) + b_ref[...]

    return pl.pallas_call(
        body,
        grid=(GRID,),
        in_specs=[
            pl.BlockSpec((CHUNK, H), lambda i: (i, 0)),
            _full((H, 4 * H)),
            _full((1, 4 * H)),
        ],
        out_specs=pl.BlockSpec((CHUNK, 4 * H), lambda i: (i, 0)),
        out_shape=jax.ShapeDtypeStruct((N, 4 * H), jnp.float32),
    )(x, WihC, bC)


def _lstm(gx, WhhC):
    T = GRID

    def body(gf_ref, gb_ref, whh_ref, hf_ref, hb_ref, st_ref):
        i = pl.program_id(0)

        @pl.when(i == 0)
        def _():
            st_ref[...] = jnp.zeros_like(st_ref)

        W = whh_ref[...]

        def step(t, carry):
            h, cc = carry
            gxf = gf_ref[pl.ds(t, 1), :]
            gxb = gb_ref[pl.ds(CHUNK - 1 - t, 1), :]
            g_all = (jnp.dot(h, W, preferred_element_type=jnp.float32)
                     + jnp.concatenate([gxf, gxb], axis=1))
            sg = 1.0 / (1.0 + jnp.exp(-g_all))
            th = jnp.tanh(g_all)
            i_cat = jnp.concatenate([sg[:, 0:64], sg[:, 256:320]], axis=1)
            f_cat = jnp.concatenate([sg[:, 64:128], sg[:, 320:384]], axis=1)
            g_cat = jnp.concatenate([th[:, 128:192], th[:, 384:448]], axis=1)
            o_cat = jnp.concatenate([sg[:, 192:256], sg[:, 448:512]], axis=1)
            cc = f_cat * cc + i_cat * g_cat
            h = o_cat * jnp.tanh(cc)
            hf_ref[pl.ds(t, 1), :] = h[:, 0:64]
            hb_ref[pl.ds(CHUNK - 1 - t, 1), :] = h[:, 64:128]
            return (h, cc)

        h0 = st_ref[0:1, :]
        c0 = st_ref[1:2, :]
        h, cc = lax.fori_loop(0, CHUNK, step, (h0, c0), unroll=8)
        st_ref[0:1, :] = h
        st_ref[1:2, :] = cc

    return pl.pallas_call(
        body,
        grid=(T,),
        in_specs=[
            pl.BlockSpec((CHUNK, 256), lambda i: (i, 0)),
            pl.BlockSpec((CHUNK, 256), lambda i: (T - 1 - i, 1)),
            _full((H, 4 * H)),
        ],
        out_specs=[
            pl.BlockSpec((CHUNK, LSTM_H), lambda i: (i, 0)),
            pl.BlockSpec((CHUNK, LSTM_H), lambda i: (T - 1 - i, 0)),
        ],
        out_shape=[
            jax.ShapeDtypeStruct((N, LSTM_H), jnp.float32),
            jax.ShapeDtypeStruct((N, LSTM_H), jnp.float32),
        ],
        scratch_shapes=[pltpu.VMEM((2, H), jnp.float32)],
        compiler_params=pltpu.CompilerParams(
            dimension_semantics=("arbitrary",)),
    )(gx, gx, WhhC)


def _heads(x3, hf, hb, WbP, bbP, WlP, blP):
    def body(x_ref, hf_ref, hb_ref, wb_ref, bb_ref, wl_ref, bl_ref,
             xc_ref, xl_ref, g_ref, b_ref, l_ref):
        i = pl.program_id(0)
        xl = jnp.concatenate([hf_ref[...], hb_ref[...]], axis=1)
        xc = x_ref[...] + xl
        xc_ref[...] = xc
        xl_ref[...] = xl

        @pl.when(i == 0)
        def _():
            g_ref[...] = jnp.zeros_like(g_ref)
            b_ref[...] = jnp.zeros_like(b_ref)
            l_ref[...] = jnp.zeros_like(l_ref)

        g_ref[...] += jnp.sum(xc, axis=0, keepdims=True)
        lane = lax.broadcasted_iota(jnp.int32, (CHUNK, H), 1)

        def probs(w, b, k):
            logits = jnp.dot(xc, w, preferred_element_type=jnp.float32) + b
            logits = jnp.where(lane < k, logits, -1e30)
            m = jnp.max(logits, axis=1, keepdims=True)
            e = jnp.where(lane < k, jnp.exp(logits - m), 0.0)
            return e / jnp.sum(e, axis=1, keepdims=True)

        b_ref[...] += jnp.sum(probs(wb_ref[...], bb_ref[...], 3),
                              axis=0, keepdims=True)
        l_ref[...] += jnp.sum(probs(wl_ref[...], bl_ref[...], 4),
                              axis=0, keepdims=True)

        @pl.when(i == GRID - 1)
        def _():
            g_ref[...] *= (1.0 / N)
            b_ref[...] *= (1.0 / N)
            l_ref[...] *= (1.0 / N)

    return pl.pallas_call(
        body,
        grid=(GRID,),
        in_specs=[
            pl.BlockSpec((CHUNK, H), lambda i: (i, 0)),
            pl.BlockSpec((CHUNK, LSTM_H), lambda i: (i, 0)),
            pl.BlockSpec((CHUNK, LSTM_H), lambda i: (i, 0)),
            _full((H, H)),
            _full((1, H)),
            _full((H, H)),
            _full((1, H)),
        ],
        out_specs=[
            pl.BlockSpec((CHUNK, H), lambda i: (i, 0)),
            pl.BlockSpec((CHUNK, H), lambda i: (i, 0)),
            pl.BlockSpec((1, H), lambda i: (0, 0)),
            pl.BlockSpec((1, H), lambda i: (0, 0)),
            pl.BlockSpec((1, H), lambda i: (0, 0)),
        ],
        out_shape=[
            jax.ShapeDtypeStruct((N, H), jnp.float32),
            jax.ShapeDtypeStruct((N, H), jnp.float32),
            jax.ShapeDtypeStruct((1, H), jnp.float32),
            jax.ShapeDtypeStruct((1, H), jnp.float32),
            jax.ShapeDtypeStruct((1, H), jnp.float32),
        ],
        compiler_params=pltpu.CompilerParams(
            dimension_semantics=("arbitrary",)),
    )(x3, hf, hb, WbP, bbP, WlP, blP)


def kernel(node_features, edge_index, block_types, W_proj, b_proj,
           block_emb_table, Wl, bl, Wr, W_ih_f, W_hh_f, b_ih_f, b_hh_f,
           W_ih_b, W_hh_b, b_ih_b, b_hh_b, W_branch, b_branch, W_loop,
           b_loop):
    f32 = jnp.float32
    src = edge_index[0]
    dst = edge_index[1]

    emb_dim = block_emb_table.shape[1]
    table_pad = jnp.pad(block_emb_table,
                        ((0, 16 - block_emb_table.shape[0]),
                         (0, H - emb_dim)))
    bt3 = block_types.reshape(GRID, CHUNK, 1)
    zeros_rp = jnp.zeros((RPT, D), f32)
    ones_rp = jnp.ones((ECH, D), f32)

    x = _proj(node_features, bt3, W_proj.T, b_proj[None], table_pad)

    agg, deg16 = _agg_deg(x, src, dst, zeros_rp, ones_rp)
    x = _layer(agg, deg16, x, Wl[0].T, Wr[0].T, bl[0][None])
    for l in range(1, Wl.shape[0]):
        agg = _agg(x, src, dst, zeros_rp)
        x = _layer(agg, deg16, x, Wl[l].T, Wr[l].T, bl[l][None])

    WihC = jnp.concatenate([W_ih_f.T, W_ih_b.T], axis=1)
    bC = jnp.concatenate([b_ih_f + b_hh_f, b_ih_b + b_hh_b])[None]
    gx = _gates(x, WihC, bC)

    WhhC = jnp.zeros((H, 4 * H), f32)
    WhhC = WhhC.at[0:LSTM_H, 0:256].set(W_hh_f.T)
    WhhC = WhhC.at[LSTM_H:H, 256:512].set(W_hh_b.T)
    hf, hb = _lstm(gx, WhhC)

    WbP = jnp.zeros((H, H), f32).at[:, 0:3].set(W_branch.T)
    bbP = jnp.zeros((1, H), f32).at[0, 0:3].set(b_branch)
    WlP = jnp.zeros((H, H), f32).at[:, 0:4].set(W_loop.T)
    blP = jnp.zeros((1, H), f32).at[0, 0:4].set(b_loop)

    xc, xl, gsum, bsum, lsum = _heads(x, hf, hb, WbP, bbP, WlP, blP)

    return (xc, gsum, bsum[:, 0:3], lsum[:, 0:4], xl)


# speedup vs baseline: 11.1480x; 1.0202x over previous
import jax
import jax.numpy as jnp
from jax import lax
from jax.experimental import pallas as pl
from jax.experimental.pallas import tpu as pltpu
from jax.experimental.pallas import tpu_sc as plsc

N = 10000
E = 320000
D = 128
H = 128
LSTM_H = 64

NC = 2
NS = 16
HN = 5120
NP = 5248
RPT = NP // NS
EPT = E // NS
ECH = 80
NCHUNK = EPT // ECH

CHUNK = 1000
GRID = N // CHUNK


NBUF = 4
NQUAD = NCHUNK // NBUF
NTAIL = NCHUNK - NQUAD * NBUF


def _make_agg(with_deg):
    out_type = [jax.ShapeDtypeStruct((NC, NS, RPT, D), jnp.float32)]
    scratch = (
        [pltpu.VMEM((ECH,), jnp.int32) for _ in range(NBUF)]
        + [pltpu.VMEM((ECH,), jnp.int32) for _ in range(NBUF)]
        + [pltpu.VMEM((ECH,), jnp.int32) for _ in range(NBUF)]
        + [pltpu.VMEM((ECH, D), jnp.float32) for _ in range(NBUF)]
        + [pltpu.VMEM((RPT, D), jnp.float32)]
        + [pltpu.VMEM_SHARED((NP, D), jnp.float32)]
        + [pltpu.SemaphoreType.DMA for _ in range(NBUF)]
    )
    if with_deg:
        out_type.append(jax.ShapeDtypeStruct((NC, NS, RPT, D), jnp.float32))

    mesh = plsc.VectorSubcoreMesh(
        core_axis_name="c", subcore_axis_name="s", num_cores=NC,
        num_subcores=NS)

    def body(*refs):
        if with_deg:
            x_hbm, src_hbm, dst_hbm, zero_hbm, ones_hbm = refs[:5]
            agg_out, deg_out = refs[5:7]
            rest = refs[7:]
        else:
            x_hbm, src_hbm, dst_hbm, zero_hbm = refs[:4]
            agg_out = refs[4]
            rest = refs[5:]
        srcv = rest[0:NBUF]
        dstv = rest[NBUF:2 * NBUF]
        dstw = rest[2 * NBUF:3 * NBUF]
        rows = rest[3 * NBUF:4 * NBUF]
        sbuf = rest[4 * NBUF]
        acc = rest[4 * NBUF + 1]
        sems = rest[4 * NBUF + 2:4 * NBUF + 2 + NBUF]

        c = lax.axis_index("c")
        s = lax.axis_index("s")
        base_node = c * HN
        rbase = s * RPT
        ebase = s * EPT

        pltpu.sync_copy(zero_hbm, sbuf)
        pltpu.sync_copy(sbuf, acc.at[pl.ds(rbase, RPT)])
        plsc.subcore_barrier()

        def remap(j, b):
            off = ebase + j * ECH
            pltpu.sync_copy(dst_hbm.at[pl.ds(off, ECH)], dstv[b])
            for k in range(ECH // 16):
                v = dstv[b][pl.ds(k * 16, 16)]
                local = v - base_node
                valid = (local >= 0) & (local < HN)
                trash = HN + ((k * 16) % (NP - HN)) + lax.iota(
                    jnp.int32, 16)
                dstw[b][pl.ds(k * 16, 16)] = jnp.where(valid, local, trash)

        def fetch(j, b):
            off = ebase + j * ECH
            remap(j, b)
            pltpu.sync_copy(src_hbm.at[pl.ds(off, ECH)], srcv[b])
            return pltpu.async_copy(x_hbm.at[srcv[b]], rows[b], sems[b])

        def drain(desc, b):
            desc.wait()
            pltpu.sync_copy(rows[b], acc.at[dstw[b]], add=True)

        def quad(i, carry):
            base = i * NBUF
            descs = [fetch(base + b, b) for b in range(NBUF)]
            for b in range(NBUF):
                drain(descs[b], b)
            return carry

        lax.fori_loop(0, NQUAD, quad, 0)
        descs = [fetch(NQUAD * NBUF + b, b) for b in range(NTAIL)]
        for b in range(NTAIL):
            drain(descs[b], b)

        plsc.subcore_barrier()
        pltpu.sync_copy(acc.at[pl.ds(rbase, RPT)], sbuf)
        pltpu.sync_copy(sbuf, agg_out.at[c, s])

        if with_deg:
            plsc.subcore_barrier()
            pltpu.sync_copy(zero_hbm, sbuf)
            pltpu.sync_copy(sbuf, acc.at[pl.ds(rbase, RPT)])
            pltpu.sync_copy(ones_hbm, rows[0])
            plsc.subcore_barrier()

            def dchunk(j, carry):
                remap(j, 1)
                pltpu.sync_copy(rows[0], acc.at[dstw[1]], add=True)
                return carry

            lax.fori_loop(0, NCHUNK, dchunk, 0)
            plsc.subcore_barrier()
            pltpu.sync_copy(acc.at[pl.ds(rbase, RPT)], sbuf)
            pltpu.sync_copy(sbuf, deg_out.at[c, s])

    return pl.kernel(body, out_type=out_type, mesh=mesh,
                     scratch_types=scratch,
                     compiler_params=pltpu.CompilerParams(
                         use_tc_tiling_on_sc=False))


def _sc_combine(p):
    q = p.reshape(NC, NP, D)
    return jnp.concatenate([q[0, :HN], q[1, : N - HN]])


def _agg_deg(x, src, dst, zeros_rp, ones_rp):
    agg_p, deg_p = _make_agg(True)(x, src, dst, zeros_rp, ones_rp)
    return _sc_combine(agg_p), _sc_combine(deg_p)[:, 0:16]


def _agg(x, src, dst, zeros_rp):
    return _sc_combine(_make_agg(False)(x, src, dst, zeros_rp)[0])


def _full(shape):
    return pl.BlockSpec(shape, lambda i: tuple(0 for _ in shape))


def _proj(nf, bt3, WpT, b2, table_pad):
    def body(nf_ref, bt_ref, wp_ref, b_ref, tp_ref, o_ref):
        x = jnp.dot(nf_ref[...], wp_ref[...],
                    preferred_element_type=jnp.float32) + b_ref[...]
        ids = bt_ref[0]
        oh = (ids == lax.broadcasted_iota(jnp.int32, (CHUNK, 16), 1)
              ).astype(jnp.float32)
        o_ref[...] = x + jnp.dot(oh, tp_ref[...],
                                 preferred_element_type=jnp.float32)

    return pl.pallas_call(
        body,
        grid=(GRID,),
        in_specs=[
            pl.BlockSpec((CHUNK, D), lambda i: (i, 0)),
            pl.BlockSpec((1, CHUNK, 1), lambda i: (i, 0, 0)),
            _full((D, H)),
            _full((1, H)),
            _full((16, H)),
        ],
        out_specs=pl.BlockSpec((CHUNK, H), lambda i: (i, 0)),
        out_shape=jax.ShapeDtypeStruct((N, H), jnp.float32),
    )(nf, bt3, WpT, b2, table_pad)


_SQRT_HALF = 0.7071067811865476


def _gelu_exact(z):
    return 0.5 * z * (1.0 + lax.erf(z * _SQRT_HALF))


def _layer(agg, deg16, x, WlT, WrT, b2):
    def body(a_ref, d_ref, x_ref, wl_ref, wr_ref, b_ref, o_ref):
        degc = d_ref[:, 0:1]
        inv = 1.0 / jnp.maximum(degc, 1.0)
        z = (jnp.dot(a_ref[...] * inv, wl_ref[...],
                     preferred_element_type=jnp.float32)
             + jnp.dot(x_ref[...], wr_ref[...],
                       preferred_element_type=jnp.float32)
             + b_ref[...])
        o_ref[...] = _gelu_exact(z)

    return pl.pallas_call(
        body,
        grid=(GRID,),
        in_specs=[
            pl.BlockSpec((CHUNK, D), lambda i: (i, 0)),
            pl.BlockSpec((CHUNK, 16), lambda i: (i, 0)),
            pl.BlockSpec((CHUNK, H), lambda i: (i, 0)),
            _full((H, H)),
            _full((H, H)),
            _full((1, H)),
        ],
        out_specs=pl.BlockSpec((CHUNK, H), lambda i: (i, 0)),
        out_shape=jax.ShapeDtypeStruct((N, H), jnp.float32),
    )(agg, deg16, x, WlT, WrT, b2)


def _gates(x, WihC, bC):
    def body(x_ref, w_ref, b_ref, o_ref):
        o_ref[...] = jnp.dot(x_ref[...], w_ref[...],
                             preferred_element_type=jnp.float32) + b_ref[...]

    return pl.pallas_call(
        body,
        grid=(GRID,),
        in_specs=[
            pl.BlockSpec((CHUNK, H), lambda i: (i, 0)),
            _full((H, 4 * H)),
            _full((1, 4 * H)),
        ],
        out_specs=pl.BlockSpec((CHUNK, 4 * H), lambda i: (i, 0)),
        out_shape=jax.ShapeDtypeStruct((N, 4 * H), jnp.float32),
    )(x, WihC, bC)


def _lstm(gx, WhhC):
    T = GRID

    def body(gf_ref, gb_ref, whh_ref, hf_ref, hb_ref, st_ref):
        i = pl.program_id(0)

        @pl.when(i == 0)
        def _():
            st_ref[...] = jnp.zeros_like(st_ref)

        W = whh_ref[...]

        def step(t, carry):
            h, cc = carry
            gxf = gf_ref[pl.ds(t, 1), :]
            gxb = gb_ref[pl.ds(CHUNK - 1 - t, 1), :]
            g_all = (jnp.dot(h, W, preferred_element_type=jnp.float32)
                     + jnp.concatenate([gxf, gxb], axis=1))
            sg = 1.0 / (1.0 + jnp.exp(-g_all))
            th = jnp.tanh(g_all)
            i_cat = jnp.concatenate([sg[:, 0:64], sg[:, 256:320]], axis=1)
            f_cat = jnp.concatenate([sg[:, 64:128], sg[:, 320:384]], axis=1)
            g_cat = jnp.concatenate([th[:, 128:192], th[:, 384:448]], axis=1)
            o_cat = jnp.concatenate([sg[:, 192:256], sg[:, 448:512]], axis=1)
            cc = f_cat * cc + i_cat * g_cat
            h = o_cat * jnp.tanh(cc)
            hf_ref[pl.ds(t, 1), :] = h[:, 0:64]
            hb_ref[pl.ds(CHUNK - 1 - t, 1), :] = h[:, 64:128]
            return (h, cc)

        h0 = st_ref[0:1, :]
        c0 = st_ref[1:2, :]
        h, cc = lax.fori_loop(0, CHUNK, step, (h0, c0), unroll=16)
        st_ref[0:1, :] = h
        st_ref[1:2, :] = cc

    return pl.pallas_call(
        body,
        grid=(T,),
        in_specs=[
            pl.BlockSpec((CHUNK, 256), lambda i: (i, 0)),
            pl.BlockSpec((CHUNK, 256), lambda i: (T - 1 - i, 1)),
            _full((H, 4 * H)),
        ],
        out_specs=[
            pl.BlockSpec((CHUNK, LSTM_H), lambda i: (i, 0)),
            pl.BlockSpec((CHUNK, LSTM_H), lambda i: (T - 1 - i, 0)),
        ],
        out_shape=[
            jax.ShapeDtypeStruct((N, LSTM_H), jnp.float32),
            jax.ShapeDtypeStruct((N, LSTM_H), jnp.float32),
        ],
        scratch_shapes=[pltpu.VMEM((2, H), jnp.float32)],
        compiler_params=pltpu.CompilerParams(
            dimension_semantics=("arbitrary",)),
    )(gx, gx, WhhC)


def _heads(x3, hf, hb, WbP, bbP, WlP, blP):
    def body(x_ref, hf_ref, hb_ref, wb_ref, bb_ref, wl_ref, bl_ref,
             xc_ref, xl_ref, g_ref, b_ref, l_ref):
        i = pl.program_id(0)
        xl = jnp.concatenate([hf_ref[...], hb_ref[...]], axis=1)
        xc = x_ref[...] + xl
        xc_ref[...] = xc
        xl_ref[...] = xl

        @pl.when(i == 0)
        def _():
            g_ref[...] = jnp.zeros_like(g_ref)
            b_ref[...] = jnp.zeros_like(b_ref)
            l_ref[...] = jnp.zeros_like(l_ref)

        g_ref[...] += jnp.sum(xc, axis=0, keepdims=True)
        lane = lax.broadcasted_iota(jnp.int32, (CHUNK, H), 1)

        def probs(w, b, k):
            logits = jnp.dot(xc, w, preferred_element_type=jnp.float32) + b
            logits = jnp.where(lane < k, logits, -1e30)
            m = jnp.max(logits, axis=1, keepdims=True)
            e = jnp.where(lane < k, jnp.exp(logits - m), 0.0)
            return e / jnp.sum(e, axis=1, keepdims=True)

        b_ref[...] += jnp.sum(probs(wb_ref[...], bb_ref[...], 3),
                              axis=0, keepdims=True)
        l_ref[...] += jnp.sum(probs(wl_ref[...], bl_ref[...], 4),
                              axis=0, keepdims=True)

        @pl.when(i == GRID - 1)
        def _():
            g_ref[...] *= (1.0 / N)
            b_ref[...] *= (1.0 / N)
            l_ref[...] *= (1.0 / N)

    return pl.pallas_call(
        body,
        grid=(GRID,),
        in_specs=[
            pl.BlockSpec((CHUNK, H), lambda i: (i, 0)),
            pl.BlockSpec((CHUNK, LSTM_H), lambda i: (i, 0)),
            pl.BlockSpec((CHUNK, LSTM_H), lambda i: (i, 0)),
            _full((H, H)),
            _full((1, H)),
            _full((H, H)),
            _full((1, H)),
        ],
        out_specs=[
            pl.BlockSpec((CHUNK, H), lambda i: (i, 0)),
            pl.BlockSpec((CHUNK, H), lambda i: (i, 0)),
            pl.BlockSpec((1, H), lambda i: (0, 0)),
            pl.BlockSpec((1, H), lambda i: (0, 0)),
            pl.BlockSpec((1, H), lambda i: (0, 0)),
        ],
        out_shape=[
            jax.ShapeDtypeStruct((N, H), jnp.float32),
            jax.ShapeDtypeStruct((N, H), jnp.float32),
            jax.ShapeDtypeStruct((1, H), jnp.float32),
            jax.ShapeDtypeStruct((1, H), jnp.float32),
            jax.ShapeDtypeStruct((1, H), jnp.float32),
        ],
        compiler_params=pltpu.CompilerParams(
            dimension_semantics=("arbitrary",)),
    )(x3, hf, hb, WbP, bbP, WlP, blP)


def kernel(node_features, edge_index, block_types, W_proj, b_proj,
           block_emb_table, Wl, bl, Wr, W_ih_f, W_hh_f, b_ih_f, b_hh_f,
           W_ih_b, W_hh_b, b_ih_b, b_hh_b, W_branch, b_branch, W_loop,
           b_loop):
    f32 = jnp.float32
    src = edge_index[0]
    dst = edge_index[1]

    emb_dim = block_emb_table.shape[1]
    table_pad = jnp.pad(block_emb_table,
                        ((0, 16 - block_emb_table.shape[0]),
                         (0, H - emb_dim)))
    bt3 = block_types.reshape(GRID, CHUNK, 1)
    zeros_rp = jnp.zeros((RPT, D), f32)
    ones_rp = jnp.ones((ECH, D), f32)

    x = _proj(node_features, bt3, W_proj.T, b_proj[None], table_pad)

    agg, deg16 = _agg_deg(x, src, dst, zeros_rp, ones_rp)
    x = _layer(agg, deg16, x, Wl[0].T, Wr[0].T, bl[0][None])
    for l in range(1, Wl.shape[0]):
        agg = _agg(x, src, dst, zeros_rp)
        x = _layer(agg, deg16, x, Wl[l].T, Wr[l].T, bl[l][None])

    WihC = jnp.concatenate([W_ih_f.T, W_ih_b.T], axis=1)
    bC = jnp.concatenate([b_ih_f + b_hh_f, b_ih_b + b_hh_b])[None]
    gx = _gates(x, WihC, bC)

    WhhC = jnp.zeros((H, 4 * H), f32)
    WhhC = WhhC.at[0:LSTM_H, 0:256].set(W_hh_f.T)
    WhhC = WhhC.at[LSTM_H:H, 256:512].set(W_hh_b.T)
    hf, hb = _lstm(gx, WhhC)

    WbP = jnp.zeros((H, H), f32).at[:, 0:3].set(W_branch.T)
    bbP = jnp.zeros((1, H), f32).at[0, 0:3].set(b_branch)
    WlP = jnp.zeros((H, H), f32).at[:, 0:4].set(W_loop.T)
    blP = jnp.zeros((1, H), f32).at[0, 0:4].set(b_loop)

    xc, xl, gsum, bsum, lsum = _heads(x, hf, hb, WbP, bbP, WlP, blP)

    return (xc, gsum, bsum[:, 0:3], lsum[:, 0:4], xl)
